# baseline (speedup 1.0000x reference)
import jax
import jax.numpy as jnp
from jax import lax
from jax.experimental import pallas as pl
from jax.experimental.pallas import tpu as pltpu
from jax.experimental.pallas import tpu_sc as plsc

BATCH = 1024
ROWS = 50
NUM_CLASSES = 1000
NC, NS, L = 2, 16, 16
NW = NC * NS
KCH = 40
NKC = NUM_CLASSES // KCH
UNITS = ROWS * NKC
GROUPS = BATCH // L
NR_PRE = 3


NBUF = 3


def _body(idx_hbm, out_hbm, buf0, buf1, buf2, idx_v, sem0, sem1, sem2):
    wid = lax.axis_index("s") * NC + lax.axis_index("c")
    u0 = wid * UNITS // NW
    u1 = (wid + 1) * UNITS // NW
    n = u1 - u0
    base_r = jnp.minimum(u0 // NKC, ROWS - NR_PRE)
    bufs = (buf0, buf1, buf2)
    sems = (sem0, sem1, sem2)

    pltpu.sync_copy(idx_hbm.at[pl.ds(base_r * BATCH, NR_PRE * BATCH)], idx_v)

    zeros = jnp.zeros((L,), jnp.float32)
    ones = jnp.ones((L,), jnp.float32)
    lane = lax.iota(jnp.int32, L)

    def zero_step(r, carry):
        for buf in bufs:
            for g in range(BATCH // L):
                buf[r, pl.ds(g * L, L)] = zeros
        return carry

    lax.fori_loop(0, KCH, zero_step, 0)

    def scatter(buf, u, vec):
        r_off = u // NKC - base_r
        k0 = (u % NKC) * KCH
        for g in range(GROUPS):
            vals = idx_v[pl.ds(r_off * BATCH + g * L, L)]
            lk = vals - k0
            in_win = (lk >= 0) & (lk < KCH)
            lk = jnp.where(in_win, lk, 0)
            plsc.store_scatter(buf, [lk, lane + g * L], vec, mask=in_win)

    def process(buf, sem, u, prev):
        @pl.when(prev >= 0)
        def _():
            pltpu.make_async_copy(buf, out_hbm.at[pl.ds(0, KCH), :], sem).wait()
            scatter(buf, prev, zeros)

        scatter(buf, u, ones)
        row0 = (u // NKC) * NUM_CLASSES + (u % NKC) * KCH
        pltpu.make_async_copy(buf, out_hbm.at[pl.ds(row0, KCH), :], sem).start()

    def ring_step(p, carry):
        prevs = list(carry)
        u_a = u0 + NBUF * p
        process(bufs[0], sems[0], u_a, prevs[0])
        prevs[0] = u_a
        for b in range(1, NBUF):
            u_b = u_a + b
            valid = u_b < u1

            @pl.when(valid)
            def _(b=b, u_b=u_b):
                process(bufs[b], sems[b], u_b, prevs[b])

            prevs[b] = jnp.where(valid, u_b, prevs[b])
        return tuple(prevs)

    lax.fori_loop(
        0, (n + NBUF - 1) // NBUF, ring_step, (jnp.int32(-1),) * NBUF
    )

    for b in range(NBUF):
        pltpu.make_async_copy(bufs[b], out_hbm.at[pl.ds(0, KCH), :], sems[b]).wait()


@jax.jit
def _one_hot_phys(idx_t):
    mesh = plsc.VectorSubcoreMesh(core_axis_name="c", subcore_axis_name="s")
    run = pl.kernel(
        _body,
        out_type=jax.ShapeDtypeStruct((ROWS * NUM_CLASSES, BATCH), jnp.float32),
        mesh=mesh,
        compiler_params=pltpu.CompilerParams(
            needs_layout_passes=False, use_tc_tiling_on_sc=True
        ),
        scratch_types=[
            pltpu.VMEM((KCH, BATCH), jnp.float32),
            pltpu.VMEM((KCH, BATCH), jnp.float32),
            pltpu.VMEM((KCH, BATCH), jnp.float32),
            pltpu.VMEM((NR_PRE * BATCH,), jnp.int32),
            pltpu.SemaphoreType.DMA,
            pltpu.SemaphoreType.DMA,
            pltpu.SemaphoreType.DMA,
        ],
    )
    return run(idx_t)


def kernel(inputs):
    idx_t = jnp.transpose(inputs).astype(jnp.int32).reshape(-1)
    out_phys = _one_hot_phys(idx_t)
    out_phys = out_phys.reshape(ROWS, NUM_CLASSES, BATCH)
    return jnp.transpose(out_phys, (2, 0, 1))


# speedup vs baseline: 1.0959x; 1.0959x over previous
import jax
import jax.numpy as jnp
from jax import lax
from jax.experimental import pallas as pl
from jax.experimental.pallas import tpu as pltpu
from jax.experimental.pallas import tpu_sc as plsc

BATCH = 1024
ROWS = 50
NUM_CLASSES = 1000
NC, NS, L = 2, 16, 16
NW = NC * NS
KCH = 40
NKC = NUM_CLASSES // KCH
UNITS = ROWS * NKC
GROUPS = BATCH // L
NR_PRE = 3


NBUF = 2


def _body(idx_hbm, out_hbm, buf0, buf1, idx_v, sem0, sem1):
    wid = lax.axis_index("s") * NC + lax.axis_index("c")
    u0 = wid * UNITS // NW
    u1 = (wid + 1) * UNITS // NW
    n = u1 - u0
    base_r = jnp.minimum(u0 // NKC, ROWS - NR_PRE)
    bufs = (buf0, buf1)
    sems = (sem0, sem1)

    pltpu.sync_copy(idx_hbm.at[pl.ds(base_r * BATCH, NR_PRE * BATCH)], idx_v)

    zeros = jnp.zeros((L,), jnp.float32)
    ones = jnp.ones((L,), jnp.float32)
    lane = lax.iota(jnp.int32, L)

    def zero_step(r, carry):
        for buf in bufs:
            for g in range(BATCH // L):
                buf[r, pl.ds(g * L, L)] = zeros
        return carry

    lax.fori_loop(0, KCH, zero_step, 0)

    def scatter(buf, u, vec):
        r_off = u // NKC - base_r
        k0 = (u % NKC) * KCH
        for g in range(GROUPS):
            vals = idx_v[pl.ds(r_off * BATCH + g * L, L)]
            lk = vals - k0
            in_win = (lk >= 0) & (lk < KCH)
            lk = jnp.where(in_win, lk, 0)
            plsc.store_scatter(buf, [lk, lane + g * L], vec, mask=in_win)

    def process(buf, sem, u, prev):
        @pl.when(prev >= 0)
        def _():
            pltpu.make_async_copy(buf, out_hbm.at[pl.ds(0, KCH), :], sem).wait()
            scatter(buf, prev, zeros)

        scatter(buf, u, ones)
        row0 = (u // NKC) * NUM_CLASSES + (u % NKC) * KCH
        pltpu.make_async_copy(buf, out_hbm.at[pl.ds(row0, KCH), :], sem).start()

    def ring_step(p, carry):
        prevs = list(carry)
        u_a = u0 + NBUF * p
        process(bufs[0], sems[0], u_a, prevs[0])
        prevs[0] = u_a
        for b in range(1, NBUF):
            u_b = u_a + b
            valid = u_b < u1

            @pl.when(valid)
            def _(b=b, u_b=u_b):
                process(bufs[b], sems[b], u_b, prevs[b])

            prevs[b] = jnp.where(valid, u_b, prevs[b])
        return tuple(prevs)

    lax.fori_loop(
        0, (n + NBUF - 1) // NBUF, ring_step, (jnp.int32(-1),) * NBUF
    )

    for b in range(NBUF):
        pltpu.make_async_copy(bufs[b], out_hbm.at[pl.ds(0, KCH), :], sems[b]).wait()


@jax.jit
def _one_hot_phys(idx_t):
    mesh = plsc.VectorSubcoreMesh(core_axis_name="c", subcore_axis_name="s")
    run = pl.kernel(
        _body,
        out_type=jax.ShapeDtypeStruct((ROWS * NUM_CLASSES, BATCH), jnp.float32),
        mesh=mesh,
        compiler_params=pltpu.CompilerParams(
            needs_layout_passes=False, use_tc_tiling_on_sc=True
        ),
        scratch_types=[
            pltpu.VMEM((KCH, BATCH), jnp.float32),
            pltpu.VMEM((KCH, BATCH), jnp.float32),
            pltpu.VMEM((NR_PRE * BATCH,), jnp.int32),
            pltpu.SemaphoreType.DMA,
            pltpu.SemaphoreType.DMA,
        ],
    )
    return run(idx_t)


def kernel(inputs):
    idx_t = jnp.transpose(inputs).astype(jnp.int32).reshape(-1)
    out_phys = _one_hot_phys(idx_t)
    out_phys = out_phys.reshape(ROWS, NUM_CLASSES, BATCH)
    return jnp.transpose(out_phys, (2, 0, 1))
